# baseline (speedup 1.0000x reference)
import functools

import jax
import jax.numpy as jnp
from jax import lax
from jax.experimental import pallas as pl
from jax.experimental.pallas import tpu as pltpu
from jax.experimental.pallas import tpu_sc as plsc

B = 16384
D = 64
NC = 2
NS = 16
NW = NC * NS
BPW = B // NW
CHUNK = 128
NCHUNK = BPW // CHUNK
L = 16
GROUPS = BPW // L


def _body(src_hbm, dst_hbm, uemb_hbm, ubias_hbm, iemb_hbm, ibias_hbm,
          mean_hbm, out_hbm,
          src_v, dst_v, bsrc_v, bdst_v, u_rows, v_rows, ub_rows, vb_rows,
          out_v, mean_v, sem):
  wid = lax.axis_index("s") * NC + lax.axis_index("c")
  base = wid * BPW

  pltpu.sync_copy(src_hbm.at[wid], src_v)
  pltpu.sync_copy(dst_hbm.at[wid], dst_v)
  pltpu.sync_copy(mean_hbm, mean_v)

  for j in range(NCHUNK):
    for t in range(CHUNK // L):
      sl = pl.ds(t * L, L)
      bsrc_v[j, sl] = lax.shift_right_logical(src_v[j, sl], 4)
      bdst_v[j, sl] = lax.shift_right_logical(dst_v[j, sl], 4)

  cps = []
  for j in range(NCHUNK):
    cps.append(pltpu.async_copy(uemb_hbm.at[src_v.at[j]], u_rows.at[j], sem))
    cps.append(pltpu.async_copy(iemb_hbm.at[dst_v.at[j]], v_rows.at[j], sem))
    cps.append(pltpu.async_copy(ubias_hbm.at[bsrc_v.at[j]], ub_rows.at[j], sem))
    cps.append(pltpu.async_copy(ibias_hbm.at[bdst_v.at[j]], vb_rows.at[j], sem))
  for cp in cps:
    cp.wait()

  lanes = lax.iota(jnp.int32, L)
  mask15 = jnp.full((L,), 15, jnp.int32)
  mean_vec = mean_v[...]

  def group(g, carry):
    chunk = g // (CHUNK // L)
    rowbase = (g % (CHUNK // L)) * L
    row = rowbase + lanes
    chunkv = jnp.zeros((L,), jnp.int32) + chunk
    acc = jnp.zeros((L,), jnp.float32)
    for d in range(D):
      col = jnp.full((L,), d, jnp.int32)
      u = plsc.load_gather(u_rows, [chunkv, row, col])
      v = plsc.load_gather(v_rows, [chunkv, row, col])
      acc = acc + u * v
    s_idx = src_v[chunk, pl.ds(rowbase, L)]
    d_idx = dst_v[chunk, pl.ds(rowbase, L)]
    ub = plsc.load_gather(ub_rows, [chunkv, row, lax.bitwise_and(s_idx, mask15)])
    vb = plsc.load_gather(vb_rows, [chunkv, row, lax.bitwise_and(d_idx, mask15)])
    out_v[pl.ds(g * L, L)] = acc + ub + vb + mean_vec
    return carry

  lax.fori_loop(0, GROUPS, group, 0)

  pltpu.sync_copy(out_v, out_hbm.at[pl.ds(base, BPW)])


@jax.jit
def kernel(src, dst, user_emb, user_bias, item_emb, item_bias, mean):
  src3 = src.astype(jnp.int32).reshape(NW, NCHUNK, CHUNK)
  dst3 = dst.astype(jnp.int32).reshape(NW, NCHUNK, CHUNK)
  ubias16 = user_bias.reshape(-1, L)
  ibias16 = item_bias.reshape(-1, L)
  mean16 = jnp.broadcast_to(mean.astype(jnp.float32), (L,))
  mesh = plsc.VectorSubcoreMesh(core_axis_name="c", subcore_axis_name="s")
  run = functools.partial(
      pl.kernel,
      out_type=jax.ShapeDtypeStruct((B,), jnp.float32),
      mesh=mesh,
      compiler_params=pltpu.CompilerParams(
          needs_layout_passes=False, use_tc_tiling_on_sc=False),
      scratch_types=[
          pltpu.VMEM((NCHUNK, CHUNK), jnp.int32),
          pltpu.VMEM((NCHUNK, CHUNK), jnp.int32),
          pltpu.VMEM((NCHUNK, CHUNK), jnp.int32),
          pltpu.VMEM((NCHUNK, CHUNK), jnp.int32),
          pltpu.VMEM((NCHUNK, CHUNK, D), jnp.float32),
          pltpu.VMEM((NCHUNK, CHUNK, D), jnp.float32),
          pltpu.VMEM((NCHUNK, CHUNK, L), jnp.float32),
          pltpu.VMEM((NCHUNK, CHUNK, L), jnp.float32),
          pltpu.VMEM((BPW,), jnp.float32),
          pltpu.VMEM((L,), jnp.float32),
          pltpu.SemaphoreType.DMA,
      ],
  )(_body)
  return run(src3, dst3, user_emb, ubias16, item_emb, ibias16, mean16)
